# baseline (speedup 1.0000x reference)
import functools

import jax
import jax.numpy as jnp
from jax import lax
from jax.experimental import pallas as pl
from jax.experimental.pallas import tpu as pltpu, tpu_sc as plsc

V = 100000
DIM = 300
H = 128
B = 64
LC = 400
LQ = 50

NC = 2
NS = 16
NW = NC * NS
N_IDS = B * LC + B * LQ
PER_W = 904
TOT = NW * PER_W
CHUNK = 128

TK = 25
RV = 2048
TK2 = 50


def _sc_gather(table, idx):
    mesh = plsc.VectorSubcoreMesh(core_axis_name="c", subcore_axis_name="s")
    n_chunks = PER_W // CHUNK
    tail = PER_W - n_chunks * CHUNK

    @functools.partial(
        pl.kernel,
        mesh=mesh,
        out_type=jax.ShapeDtypeStruct((TOT, H), jnp.float32),
        scratch_types=[
            pltpu.VMEM((PER_W,), jnp.int32),
            pltpu.VMEM((CHUNK, H), jnp.float32),
            pltpu.VMEM((CHUNK, H), jnp.float32),
            pltpu.SemaphoreType.DMA,
            pltpu.SemaphoreType.DMA,
        ],
    )
    def k(table_hbm, idx_hbm, out_hbm, idx_v, buf0, buf1, sem0, sem1):
        wid = lax.axis_index("s") * NC + lax.axis_index("c")
        base = wid * PER_W
        pltpu.sync_copy(idx_hbm.at[pl.ds(base, PER_W)], idx_v)
        sizes = [CHUNK] * n_chunks + ([tail] if tail else [])
        offs = [i * CHUNK for i in range(len(sizes))]
        bufs = (buf0, buf1)
        sems = (sem0, sem1)

        def start(j):
            return pltpu.async_copy(
                table_hbm.at[idx_v.at[pl.ds(offs[j], sizes[j])]],
                bufs[j % 2].at[pl.ds(0, sizes[j])],
                sems[j % 2],
            )

        cps = [None] * len(sizes)
        cps[0] = start(0)
        for j in range(len(sizes)):
            if j + 1 < len(sizes):
                cps[j + 1] = start(j + 1)
            cps[j].wait()
            pltpu.sync_copy(
                bufs[j % 2].at[pl.ds(0, sizes[j])],
                out_hbm.at[pl.ds(base + offs[j], sizes[j])],
            )

    return k(table, idx)


def _dot_t(x, w):
    return lax.dot_general(x, w, (((1,), (1,)), ((), ())),
                           preferred_element_type=jnp.float32)


def _dot_t16(x, w):
    return lax.dot_general(x.astype(jnp.bfloat16), w.astype(jnp.bfloat16),
                           (((1,), (1,)), ((), ())),
                           preferred_element_type=jnp.float32)


def _dot16(x, wt):
    return lax.dot_general(x.astype(jnp.bfloat16), wt.astype(jnp.bfloat16),
                           (((1,), (0,)), ((), ())),
                           preferred_element_type=jnp.float32)


def _dot_tl16(xt, w):
    return lax.dot_general(xt.astype(jnp.bfloat16), w.astype(jnp.bfloat16),
                           (((0,), (1,)), ((), ())),
                           preferred_element_type=jnp.float32)


def _tablefwd_body(x_ref, pw_ref, wg0_ref, bg0_ref, wt0_ref, bt0_ref,
                   wg1_ref, bg1_ref, wt1_ref, bt1_ref, out_ref):
    x = x_ref[...]
    e = _dot_tl16(x, pw_ref[...])
    for wg_ref, bg_ref, wt_ref, bt_ref in (
        (wg0_ref, bg0_ref, wt0_ref, bt0_ref),
        (wg1_ref, bg1_ref, wt1_ref, bt1_ref),
    ):
        g = jax.nn.sigmoid(_dot_t16(e, wg_ref[...]) + bg_ref[0])
        t = jnp.maximum(_dot_t16(e, wt_ref[...]) + bt_ref[0], 0.0)
        e = g * t + (1.0 - g) * e
    ep = e.reshape(RV // 2, 2, H)
    ue = lax.shift_right_logical(
        lax.bitcast_convert_type(ep[:, 0, :].astype(jnp.bfloat16)
                                 .astype(jnp.float32), jnp.uint32),
        jnp.uint32(16))
    uo = lax.bitcast_convert_type(ep[:, 1, :].astype(jnp.bfloat16)
                                  .astype(jnp.float32), jnp.uint32)
    uo = uo & jnp.uint32(0xFFFF0000)
    out_ref[...] = lax.bitcast_convert_type(uo | ue, jnp.float32)


def _tablefwd(wv_t, proj_w, wg0, bg0, wt0, bt0, wg1, bg1, wt1, bt1):
    grid = (V + RV - 1) // RV
    full = lambda shape: pl.BlockSpec(shape, lambda i: (0,) * len(shape))
    return pl.pallas_call(
        _tablefwd_body,
        grid=(grid,),
        in_specs=[
            pl.BlockSpec((DIM, RV), lambda i: (0, i)),
            full((H, DIM)),
            full((H, H)), full((1, H)), full((H, H)), full((1, H)),
            full((H, H)), full((1, H)), full((H, H)), full((1, H)),
        ],
        out_specs=pl.BlockSpec((RV // 2, H), lambda i: (i, 0)),
        out_shape=jax.ShapeDtypeStruct((V // 2, H), jnp.float32),
    )(wv_t, proj_w, wg0, bg0, wt0, bt0, wg1, bg1, wt1, bt1)


def _unpack(xp, par_ref):
    u = lax.bitcast_convert_type(xp, jnp.uint32)
    lo = lax.bitcast_convert_type(lax.shift_left(u, jnp.uint32(16)), jnp.float32)
    hi = lax.bitcast_convert_type(u & jnp.uint32(0xFFFF0000), jnp.float32)
    return jnp.where(par_ref[...] == 1, hi, lo)


def _lstm1_body(cx_ref, qx_ref, pc_ref, pq_ref, wih_ref, whh_ref, b_ref,
                qw_ref, qb_ref, lens_ref, outc_ref, outq_ref, hout_ref, cout_ref,
                h_ref, c_ref):
    j = pl.program_id(0)

    @pl.when(j == 0)
    def _():
        h_ref[...] = jnp.zeros_like(h_ref)
        c_ref[...] = jnp.zeros_like(c_ref)

    h = h_ref[...]
    c = c_ref[...]
    whh = whh_ref[...]
    lens = lens_ref[...]

    xc = _unpack(cx_ref[...], pc_ref).reshape(TK, B, H)
    xq = _unpack(qx_ref[...], pq_ref).reshape(TK, B, H)
    x = jnp.concatenate([xc, xq], axis=1)
    gx = _dot_t16(x.reshape(TK * 2 * B, H), wih_ref[...]) + b_ref[0]
    gx = gx.reshape(TK, 2 * B, 4 * H)

    def sig(v):
        return 0.5 + 0.5 * jnp.tanh(0.5 * v)

    for s in range(TK):
        t = j * TK + s
        gates = gx[s] + _dot_t16(h, whh)
        i_g = sig(gates[:, 0 * H:1 * H])
        f_g = sig(gates[:, 1 * H:2 * H])
        g_g = jnp.tanh(gates[:, 2 * H:3 * H])
        o_g = sig(gates[:, 3 * H:4 * H])
        c = f_g * c + i_g * g_g
        h = o_g * jnp.tanh(c)
        m = (lens > t).astype(jnp.float32)
        hm = h * m
        outc_ref[s] = hm[:B]
        qh = jnp.tanh(_dot_t16(hm[B:], qw_ref[...]) + qb_ref[0])
        outq_ref[pl.ds(t, 1)] = qh[None]

    h_ref[...] = h
    c_ref[...] = c
    hout_ref[...] = h
    cout_ref[...] = c


def _lstm2_body(acc_ref, cx_ref, pc_ref, wih_ref, whh_ref, b_ref, lens_ref,
                h0_ref, c0_ref, outc_ref, h_ref, c_ref):
    del acc_ref
    j = pl.program_id(0)

    @pl.when(j == 0)
    def _():
        h_ref[...] = h0_ref[...]
        c_ref[...] = c0_ref[...]

    h = h_ref[...]
    c = c_ref[...]
    whh = whh_ref[...]
    lens = lens_ref[...]

    gx = _dot16(_unpack(cx_ref[...], pc_ref), wih_ref[...]) + b_ref[0]
    gx = gx.reshape(TK2, B, 4 * H)

    def sig(v):
        return 0.5 + 0.5 * jnp.tanh(0.5 * v)

    HB = B // 2
    ha, hb = h[:HB], h[HB:]
    ca, cb = c[:HB], c[HB:]

    def step(gx_s, h_half, c_half):
        gates = gx_s + _dot16(h_half, whh)
        i_g = sig(gates[:, 0 * H:1 * H])
        f_g = sig(gates[:, 1 * H:2 * H])
        g_g = jnp.tanh(gates[:, 2 * H:3 * H])
        o_g = sig(gates[:, 3 * H:4 * H])
        c_half = f_g * c_half + i_g * g_g
        h_half = o_g * jnp.tanh(c_half)
        return h_half, c_half

    for s in range(TK2):
        t = LQ + j * TK2 + s
        ha, ca = step(gx[s, :HB], ha, ca)
        hb, cb = step(gx[s, HB:], hb, cb)
        m = (lens > t).astype(jnp.float32)
        outc_ref[s] = jnp.concatenate([ha, hb], axis=0) * m

    h_ref[...] = jnp.concatenate([ha, hb], axis=0)
    c_ref[...] = jnp.concatenate([ca, cb], axis=0)


def _lstm(emb, par, w_ih, w_hh, b, qproj_w, qproj_b, lens):
    RB = TK * B
    full = lambda shape: pl.BlockSpec(shape, lambda j: (0,) * len(shape))
    out_c1, out_q, h1, c1 = pl.pallas_call(
        _lstm1_body,
        grid=(LQ // TK,),
        in_specs=[
            pl.BlockSpec((RB, H), lambda j: (j, 0)),
            pl.BlockSpec((RB, H), lambda j: (LC // TK + j, 0)),
            pl.BlockSpec((RB, 1), lambda j: (j, 0)),
            pl.BlockSpec((RB, 1), lambda j: (LC // TK + j, 0)),
            full((4 * H, H)),
            full((4 * H, H)),
            full((1, 4 * H)),
            full((H, H)),
            full((1, H)),
            full((2 * B, 1)),
        ],
        out_specs=[
            pl.BlockSpec((TK, B, H), lambda j: (j, 0, 0)),
            pl.BlockSpec((LQ, B, H), lambda j: (0, 0, 0)),
            full((2 * B, H)),
            full((2 * B, H)),
        ],
        out_shape=[
            jax.ShapeDtypeStruct((LC + 1, B, H), jnp.float32),
            jax.ShapeDtypeStruct((LQ, B, H), jnp.float32),
            jax.ShapeDtypeStruct((2 * B, H), jnp.float32),
            jax.ShapeDtypeStruct((2 * B, H), jnp.float32),
        ],
        scratch_shapes=[
            pltpu.VMEM((2 * B, H), jnp.float32),
            pltpu.VMEM((2 * B, H), jnp.float32),
        ],
        compiler_params=pltpu.CompilerParams(
            dimension_semantics=("arbitrary",),
        ),
    )(emb, emb, par, par, w_ih, w_hh, b, qproj_w, qproj_b, lens)

    RB2 = TK2 * B
    out_c = pl.pallas_call(
        _lstm2_body,
        grid=((LC - LQ) // TK2,),
        in_specs=[
            pl.BlockSpec(memory_space=pl.ANY),
            pl.BlockSpec((RB2, H), lambda j: (LQ // TK2 + j, 0)),
            pl.BlockSpec((RB2, 1), lambda j: (LQ // TK2 + j, 0)),
            full((H, 4 * H)),
            full((H, 4 * H)),
            full((1, 4 * H)),
            full((B, 1)),
            pl.BlockSpec((B, H), lambda j: (0, 0)),
            pl.BlockSpec((B, H), lambda j: (0, 0)),
        ],
        out_specs=pl.BlockSpec((TK2, B, H), lambda j: (j + 1, 0, 0)),
        out_shape=jax.ShapeDtypeStruct((LC + 1, B, H), jnp.float32),
        scratch_shapes=[
            pltpu.VMEM((B, H), jnp.float32),
            pltpu.VMEM((B, H), jnp.float32),
        ],
        input_output_aliases={0: 0},
        compiler_params=pltpu.CompilerParams(
            dimension_semantics=("arbitrary",),
        ),
    )(out_c1, emb, par, w_ih.T, w_hh.T, b, lens[:B], h1, c1)

    return out_c, out_q


def _sentinel_body(acc_ref, s_ref, out_ref):
    del acc_ref
    out_ref[...] = jnp.broadcast_to(s_ref[0], (1, B, H))


def _append_sentinel(out_c, sentinel):
    return pl.pallas_call(
        _sentinel_body,
        grid=(1,),
        in_specs=[
            pl.BlockSpec(memory_space=pl.ANY),
            pl.BlockSpec((1, H), lambda i: (0, 0)),
        ],
        out_specs=pl.BlockSpec((1, B, H), lambda i: (LC, 0, 0)),
        out_shape=jax.ShapeDtypeStruct((LC + 1, B, H), jnp.float32),
        input_output_aliases={0: 0},
    )(out_c, sentinel)


def kernel(cw_idxs, qw_idxs, word_vectors, proj_w, hwy_wg0, hwy_bg0,
           hwy_wt0, hwy_bt0, hwy_wg1, hwy_bg1, hwy_wt1, hwy_bt1,
           lstm_w_ih, lstm_w_hh, lstm_b, qproj_w, qproj_b,
           sentinel_c, sentinel_q):
    idx_c = cw_idxs.T.reshape(-1).astype(jnp.int32)
    idx_q = qw_idxs.T.reshape(-1).astype(jnp.int32)
    pad = jnp.zeros((TOT - N_IDS,), jnp.int32)
    idx_all = jnp.concatenate([idx_c, idx_q, pad])

    table = _tablefwd(word_vectors.T, proj_w,
                      hwy_wg0, hwy_bg0.reshape(1, H), hwy_wt0, hwy_bt0.reshape(1, H),
                      hwy_wg1, hwy_bg1.reshape(1, H), hwy_wt1, hwy_bt1.reshape(1, H))

    emb = _sc_gather(table, idx_all // 2)
    par = (idx_all & 1).reshape(TOT, 1)

    c_len = (cw_idxs != 0).sum(-1).astype(jnp.int32)
    q_len = (qw_idxs != 0).sum(-1).astype(jnp.int32)
    lens = jnp.concatenate([c_len, q_len]).reshape(2 * B, 1)

    b2 = lstm_b.reshape(1, 4 * H)
    qb2 = qproj_b.reshape(1, H)
    out_c, out_q = _lstm(emb, par, lstm_w_ih, lstm_w_hh, b2,
                         qproj_w, qb2, lens)

    out_d = _append_sentinel(out_c, sentinel_c.reshape(1, H))
    D = jnp.transpose(out_d, (1, 0, 2))
    sq = jnp.broadcast_to(sentinel_q[None, None, :], (B, 1, H))
    Q = jnp.concatenate([jnp.transpose(out_q, (1, 0, 2)), sq], axis=1)
    return (D, Q)


# speedup vs baseline: 1.2338x; 1.2338x over previous
import functools

import jax
import jax.numpy as jnp
from jax import lax
from jax.experimental import pallas as pl
from jax.experimental.pallas import tpu as pltpu, tpu_sc as plsc

V = 100000
DIM = 300
H = 128
B = 64
LC = 400
LQ = 50

NC = 2
NS = 16
NW = NC * NS
N_IDS = B * LC + B * LQ
PER_W = 904
TOT = NW * PER_W
CHUNK = 128

TK = 25
RV = 2048
TK2 = 50


def _sc_gather(table, idx):
    mesh = plsc.VectorSubcoreMesh(core_axis_name="c", subcore_axis_name="s")
    n_chunks = PER_W // CHUNK
    tail = PER_W - n_chunks * CHUNK

    @functools.partial(
        pl.kernel,
        mesh=mesh,
        out_type=jax.ShapeDtypeStruct((TOT, H), jnp.float32),
        scratch_types=[
            pltpu.VMEM((PER_W,), jnp.int32),
            pltpu.VMEM((CHUNK, H), jnp.float32),
            pltpu.VMEM((CHUNK, H), jnp.float32),
            pltpu.SemaphoreType.DMA,
            pltpu.SemaphoreType.DMA,
        ],
    )
    def k(table_hbm, idx_hbm, out_hbm, idx_v, buf0, buf1, sem0, sem1):
        wid = lax.axis_index("s") * NC + lax.axis_index("c")
        base = wid * PER_W
        pltpu.sync_copy(idx_hbm.at[pl.ds(base, PER_W)], idx_v)
        sizes = [CHUNK] * n_chunks + ([tail] if tail else [])
        offs = [i * CHUNK for i in range(len(sizes))]
        bufs = (buf0, buf1)
        sems = (sem0, sem1)

        def start(j):
            return pltpu.async_copy(
                table_hbm.at[idx_v.at[pl.ds(offs[j], sizes[j])]],
                bufs[j % 2].at[pl.ds(0, sizes[j])],
                sems[j % 2],
            )

        cps = [None] * len(sizes)
        cps[0] = start(0)
        for j in range(len(sizes)):
            if j + 1 < len(sizes):
                cps[j + 1] = start(j + 1)
            cps[j].wait()
            pltpu.sync_copy(
                bufs[j % 2].at[pl.ds(0, sizes[j])],
                out_hbm.at[pl.ds(base + offs[j], sizes[j])],
            )

    return k(table, idx)


def _dot_t(x, w):
    return lax.dot_general(x, w, (((1,), (1,)), ((), ())),
                           preferred_element_type=jnp.float32)


def _dot_t16(x, w):
    return lax.dot_general(x.astype(jnp.bfloat16), w.astype(jnp.bfloat16),
                           (((1,), (1,)), ((), ())),
                           preferred_element_type=jnp.float32)


def _dot16(x, wt):
    return lax.dot_general(x.astype(jnp.bfloat16), wt.astype(jnp.bfloat16),
                           (((1,), (0,)), ((), ())),
                           preferred_element_type=jnp.float32)


def _dot_tl16(xt, w):
    return lax.dot_general(xt.astype(jnp.bfloat16), w.astype(jnp.bfloat16),
                           (((0,), (1,)), ((), ())),
                           preferred_element_type=jnp.float32)


def _tablefwd_body(x_ref, pw_ref, wg0_ref, bg0_ref, wt0_ref, bt0_ref,
                   wg1_ref, bg1_ref, wt1_ref, bt1_ref, out_ref):
    x = x_ref[...]
    e = _dot_tl16(x, pw_ref[...])
    for wg_ref, bg_ref, wt_ref, bt_ref in (
        (wg0_ref, bg0_ref, wt0_ref, bt0_ref),
        (wg1_ref, bg1_ref, wt1_ref, bt1_ref),
    ):
        g = jax.nn.sigmoid(_dot_t16(e, wg_ref[...]) + bg_ref[0])
        t = jnp.maximum(_dot_t16(e, wt_ref[...]) + bt_ref[0], 0.0)
        e = g * t + (1.0 - g) * e
    out_ref[...] = e


def _tablefwd(wv_t, proj_w, wg0, bg0, wt0, bt0, wg1, bg1, wt1, bt1):
    grid = (V + RV - 1) // RV
    full = lambda shape: pl.BlockSpec(shape, lambda i: (0,) * len(shape))
    return pl.pallas_call(
        _tablefwd_body,
        grid=(grid,),
        in_specs=[
            pl.BlockSpec((DIM, RV), lambda i: (0, i)),
            full((H, DIM)),
            full((H, H)), full((1, H)), full((H, H)), full((1, H)),
            full((H, H)), full((1, H)), full((H, H)), full((1, H)),
        ],
        out_specs=pl.BlockSpec((RV, H), lambda i: (i, 0)),
        out_shape=jax.ShapeDtypeStruct((V, H), jnp.float32),
    )(wv_t, proj_w, wg0, bg0, wt0, bt0, wg1, bg1, wt1, bt1)


def _lstm1_body(cx_ref, qx_ref, wih_ref, whh_ref, b_ref, qw_ref, qb_ref,
                lens_ref, outc_ref, outq_ref, hout_ref, cout_ref,
                h_ref, c_ref):
    j = pl.program_id(0)

    @pl.when(j == 0)
    def _():
        h_ref[...] = jnp.zeros_like(h_ref)
        c_ref[...] = jnp.zeros_like(c_ref)

    h = h_ref[...]
    c = c_ref[...]
    whh = whh_ref[...]
    lens = lens_ref[...]

    x = jnp.concatenate([cx_ref[...].reshape(TK, B, H),
                         qx_ref[...].reshape(TK, B, H)], axis=1)
    gx = _dot_t16(x.reshape(TK * 2 * B, H), wih_ref[...]) + b_ref[0]
    gx = gx.reshape(TK, 2 * B, 4 * H)

    def sig(v):
        return 0.5 + 0.5 * jnp.tanh(0.5 * v)

    for s in range(TK):
        t = j * TK + s
        gates = gx[s] + _dot_t16(h, whh)
        i_g = sig(gates[:, 0 * H:1 * H])
        f_g = sig(gates[:, 1 * H:2 * H])
        g_g = jnp.tanh(gates[:, 2 * H:3 * H])
        o_g = sig(gates[:, 3 * H:4 * H])
        c = f_g * c + i_g * g_g
        h = o_g * jnp.tanh(c)
        m = (lens > t).astype(jnp.float32)
        hm = h * m
        outc_ref[s] = hm[:B]
        qh = jnp.tanh(_dot_t16(hm[B:], qw_ref[...]) + qb_ref[0])
        outq_ref[pl.ds(t, 1)] = qh[None]

    h_ref[...] = h
    c_ref[...] = c
    hout_ref[...] = h
    cout_ref[...] = c


def _lstm2_body(acc_ref, cx_ref, wih_ref, whh_ref, b_ref, lens_ref,
                h0_ref, c0_ref, outc_ref, h_ref, c_ref):
    del acc_ref
    j = pl.program_id(0)

    @pl.when(j == 0)
    def _():
        h_ref[...] = h0_ref[...]
        c_ref[...] = c0_ref[...]

    h = h_ref[...]
    c = c_ref[...]
    whh = whh_ref[...]
    lens = lens_ref[...]

    gx = _dot16(cx_ref[...], wih_ref[...]) + b_ref[0]
    gx = gx.reshape(TK2, B, 4 * H)

    def sig(v):
        return 0.5 + 0.5 * jnp.tanh(0.5 * v)

    HB = B // 2
    ha, hb = h[:HB], h[HB:]
    ca, cb = c[:HB], c[HB:]

    def step(gx_s, h_half, c_half):
        gates = gx_s + _dot16(h_half, whh)
        i_g = sig(gates[:, 0 * H:1 * H])
        f_g = sig(gates[:, 1 * H:2 * H])
        g_g = jnp.tanh(gates[:, 2 * H:3 * H])
        o_g = sig(gates[:, 3 * H:4 * H])
        c_half = f_g * c_half + i_g * g_g
        h_half = o_g * jnp.tanh(c_half)
        return h_half, c_half

    for s in range(TK2):
        t = LQ + j * TK2 + s
        ha, ca = step(gx[s, :HB], ha, ca)
        hb, cb = step(gx[s, HB:], hb, cb)
        m = (lens > t).astype(jnp.float32)
        outc_ref[s] = jnp.concatenate([ha, hb], axis=0) * m

    h_ref[...] = jnp.concatenate([ha, hb], axis=0)
    c_ref[...] = jnp.concatenate([ca, cb], axis=0)


def _lstm(emb, w_ih, w_hh, b, qproj_w, qproj_b, lens):
    RB = TK * B
    full = lambda shape: pl.BlockSpec(shape, lambda j: (0,) * len(shape))
    out_c1, out_q, h1, c1 = pl.pallas_call(
        _lstm1_body,
        grid=(LQ // TK,),
        in_specs=[
            pl.BlockSpec((RB, H), lambda j: (j, 0)),
            pl.BlockSpec((RB, H), lambda j: (LC // TK + j, 0)),
            full((4 * H, H)),
            full((4 * H, H)),
            full((1, 4 * H)),
            full((H, H)),
            full((1, H)),
            full((2 * B, 1)),
        ],
        out_specs=[
            pl.BlockSpec((TK, B, H), lambda j: (j, 0, 0)),
            pl.BlockSpec((LQ, B, H), lambda j: (0, 0, 0)),
            full((2 * B, H)),
            full((2 * B, H)),
        ],
        out_shape=[
            jax.ShapeDtypeStruct((LC + 1, B, H), jnp.float32),
            jax.ShapeDtypeStruct((LQ, B, H), jnp.float32),
            jax.ShapeDtypeStruct((2 * B, H), jnp.float32),
            jax.ShapeDtypeStruct((2 * B, H), jnp.float32),
        ],
        scratch_shapes=[
            pltpu.VMEM((2 * B, H), jnp.float32),
            pltpu.VMEM((2 * B, H), jnp.float32),
        ],
        compiler_params=pltpu.CompilerParams(
            dimension_semantics=("arbitrary",),
        ),
    )(emb, emb, w_ih, w_hh, b, qproj_w, qproj_b, lens)

    RB2 = TK2 * B
    out_c = pl.pallas_call(
        _lstm2_body,
        grid=((LC - LQ) // TK2,),
        in_specs=[
            pl.BlockSpec(memory_space=pl.ANY),
            pl.BlockSpec((RB2, H), lambda j: (LQ // TK2 + j, 0)),
            full((H, 4 * H)),
            full((H, 4 * H)),
            full((1, 4 * H)),
            full((B, 1)),
            pl.BlockSpec((B, H), lambda j: (0, 0)),
            pl.BlockSpec((B, H), lambda j: (0, 0)),
        ],
        out_specs=pl.BlockSpec((TK2, B, H), lambda j: (j + 1, 0, 0)),
        out_shape=jax.ShapeDtypeStruct((LC + 1, B, H), jnp.float32),
        scratch_shapes=[
            pltpu.VMEM((B, H), jnp.float32),
            pltpu.VMEM((B, H), jnp.float32),
        ],
        input_output_aliases={0: 0},
        compiler_params=pltpu.CompilerParams(
            dimension_semantics=("arbitrary",),
        ),
    )(out_c1, emb, w_ih.T, w_hh.T, b, lens[:B], h1, c1)

    return out_c, out_q


def _sentinel_body(acc_ref, s_ref, out_ref):
    del acc_ref
    out_ref[...] = jnp.broadcast_to(s_ref[0], (1, B, H))


def _append_sentinel(out_c, sentinel):
    return pl.pallas_call(
        _sentinel_body,
        grid=(1,),
        in_specs=[
            pl.BlockSpec(memory_space=pl.ANY),
            pl.BlockSpec((1, H), lambda i: (0, 0)),
        ],
        out_specs=pl.BlockSpec((1, B, H), lambda i: (LC, 0, 0)),
        out_shape=jax.ShapeDtypeStruct((LC + 1, B, H), jnp.float32),
        input_output_aliases={0: 0},
    )(out_c, sentinel)


def kernel(cw_idxs, qw_idxs, word_vectors, proj_w, hwy_wg0, hwy_bg0,
           hwy_wt0, hwy_bt0, hwy_wg1, hwy_bg1, hwy_wt1, hwy_bt1,
           lstm_w_ih, lstm_w_hh, lstm_b, qproj_w, qproj_b,
           sentinel_c, sentinel_q):
    idx_c = cw_idxs.T.reshape(-1).astype(jnp.int32)
    idx_q = qw_idxs.T.reshape(-1).astype(jnp.int32)
    pad = jnp.zeros((TOT - N_IDS,), jnp.int32)
    idx_all = jnp.concatenate([idx_c, idx_q, pad])

    table = _tablefwd(word_vectors.T, proj_w,
                      hwy_wg0, hwy_bg0.reshape(1, H), hwy_wt0, hwy_bt0.reshape(1, H),
                      hwy_wg1, hwy_bg1.reshape(1, H), hwy_wt1, hwy_bt1.reshape(1, H))

    emb = _sc_gather(table, idx_all)

    c_len = (cw_idxs != 0).sum(-1).astype(jnp.int32)
    q_len = (qw_idxs != 0).sum(-1).astype(jnp.int32)
    lens = jnp.concatenate([c_len, q_len]).reshape(2 * B, 1)

    b2 = lstm_b.reshape(1, 4 * H)
    qb2 = qproj_b.reshape(1, H)
    out_c, out_q = _lstm(emb, lstm_w_ih, lstm_w_hh, b2,
                         qproj_w, qb2, lens)

    out_d = _append_sentinel(out_c, sentinel_c.reshape(1, H))
    D = jnp.transpose(out_d, (1, 0, 2))
    sq = jnp.broadcast_to(sentinel_q[None, None, :], (B, 1, H))
    Q = jnp.concatenate([jnp.transpose(out_q, (1, 0, 2)), sq], axis=1)
    return (D, Q)


# speedup vs baseline: 1.2982x; 1.0522x over previous
import functools

import jax
import jax.numpy as jnp
from jax import lax
from jax.experimental import pallas as pl
from jax.experimental.pallas import tpu as pltpu, tpu_sc as plsc

V = 100000
DIM = 300
H = 128
B = 64
LC = 400
LQ = 50

NC = 2
NS = 16
NW = NC * NS
N_IDS = B * LC + B * LQ
PER_W = 904
TOT = NW * PER_W
CHUNK = 128

TK = 25
RV = 2048
TK2 = 50


def _sc_gather(table, idx, per_w, sizes):
    mesh = plsc.VectorSubcoreMesh(core_axis_name="c", subcore_axis_name="s")
    offs = [sum(sizes[:i]) for i in range(len(sizes))]

    @functools.partial(
        pl.kernel,
        mesh=mesh,
        out_type=jax.ShapeDtypeStruct((NW * per_w, H), jnp.float32),
        scratch_types=[
            pltpu.VMEM((per_w,), jnp.int32),
            pltpu.VMEM((CHUNK, H), jnp.float32),
            pltpu.VMEM((CHUNK, H), jnp.float32),
            pltpu.SemaphoreType.DMA,
            pltpu.SemaphoreType.DMA,
        ],
    )
    def k(table_hbm, idx_hbm, out_hbm, idx_v, buf0, buf1, sem0, sem1):
        wid = lax.axis_index("s") * NC + lax.axis_index("c")
        base = wid * per_w
        pltpu.sync_copy(idx_hbm.at[pl.ds(base, per_w)], idx_v)
        bufs = (buf0, buf1)
        sems = (sem0, sem1)

        def start(j):
            return pltpu.async_copy(
                table_hbm.at[idx_v.at[pl.ds(offs[j], sizes[j])]],
                bufs[j % 2].at[pl.ds(0, sizes[j])],
                sems[j % 2],
            )

        cps = [None] * len(sizes)
        cps[0] = start(0)
        for j in range(len(sizes)):
            if j + 1 < len(sizes):
                cps[j + 1] = start(j + 1)
            cps[j].wait()
            pltpu.sync_copy(
                bufs[j % 2].at[pl.ds(0, sizes[j])],
                out_hbm.at[pl.ds(base + offs[j], sizes[j])],
            )

    return k(table, idx)


def _dot_t(x, w):
    return lax.dot_general(x, w, (((1,), (1,)), ((), ())),
                           preferred_element_type=jnp.float32)


def _dot_t16(x, w):
    return lax.dot_general(x.astype(jnp.bfloat16), w.astype(jnp.bfloat16),
                           (((1,), (1,)), ((), ())),
                           preferred_element_type=jnp.float32)


def _dot16(x, wt):
    return lax.dot_general(x.astype(jnp.bfloat16), wt.astype(jnp.bfloat16),
                           (((1,), (0,)), ((), ())),
                           preferred_element_type=jnp.float32)


def _dot_tl16(xt, w):
    return lax.dot_general(xt.astype(jnp.bfloat16), w.astype(jnp.bfloat16),
                           (((0,), (1,)), ((), ())),
                           preferred_element_type=jnp.float32)


def _tablefwd_body(x_ref, pw_ref, wg0_ref, bg0_ref, wt0_ref, bt0_ref,
                   wg1_ref, bg1_ref, wt1_ref, bt1_ref, out_ref):
    x = x_ref[...]
    e = _dot_tl16(x, pw_ref[...])
    for wg_ref, bg_ref, wt_ref, bt_ref in (
        (wg0_ref, bg0_ref, wt0_ref, bt0_ref),
        (wg1_ref, bg1_ref, wt1_ref, bt1_ref),
    ):
        g = jax.nn.sigmoid(_dot_t16(e, wg_ref[...]) + bg_ref[0])
        t = jnp.maximum(_dot_t16(e, wt_ref[...]) + bt_ref[0], 0.0)
        e = g * t + (1.0 - g) * e
    out_ref[...] = e


def _tablefwd(wv_t, proj_w, wg0, bg0, wt0, bt0, wg1, bg1, wt1, bt1):
    grid = (V + RV - 1) // RV
    full = lambda shape: pl.BlockSpec(shape, lambda i: (0,) * len(shape))
    return pl.pallas_call(
        _tablefwd_body,
        grid=(grid,),
        in_specs=[
            pl.BlockSpec((DIM, RV), lambda i: (0, i)),
            full((H, DIM)),
            full((H, H)), full((1, H)), full((H, H)), full((1, H)),
            full((H, H)), full((1, H)), full((H, H)), full((1, H)),
        ],
        out_specs=pl.BlockSpec((RV, H), lambda i: (i, 0)),
        out_shape=jax.ShapeDtypeStruct((V, H), jnp.float32),
    )(wv_t, proj_w, wg0, bg0, wt0, bt0, wg1, bg1, wt1, bt1)


def _lstm1_body(cx_ref, qx_ref, wih_ref, whh_ref, b_ref, qw_ref, qb_ref,
                lens_ref, outc_ref, outq_ref, hout_ref, cout_ref,
                h_ref, c_ref):
    j = pl.program_id(0)

    @pl.when(j == 0)
    def _():
        h_ref[...] = jnp.zeros_like(h_ref)
        c_ref[...] = jnp.zeros_like(c_ref)

    h = h_ref[...]
    c = c_ref[...]
    whh = whh_ref[...]
    lens = lens_ref[...]

    x = jnp.concatenate([cx_ref[...].reshape(TK, B, H),
                         qx_ref[...].reshape(TK, B, H)], axis=1)
    gx = _dot_t16(x.reshape(TK * 2 * B, H), wih_ref[...]) + b_ref[0]
    gx = gx.reshape(TK, 2 * B, 4 * H)

    def sig(v):
        return 0.5 + 0.5 * jnp.tanh(0.5 * v)

    for s in range(TK):
        t = j * TK + s
        gates = gx[s] + _dot_t16(h, whh)
        i_g = sig(gates[:, 0 * H:1 * H])
        f_g = sig(gates[:, 1 * H:2 * H])
        g_g = jnp.tanh(gates[:, 2 * H:3 * H])
        o_g = sig(gates[:, 3 * H:4 * H])
        c = f_g * c + i_g * g_g
        h = o_g * jnp.tanh(c)
        m = (lens > t).astype(jnp.float32)
        hm = h * m
        outc_ref[s] = hm[:B]
        qh = jnp.tanh(_dot_t16(hm[B:], qw_ref[...]) + qb_ref[0])
        outq_ref[pl.ds(t, 1)] = qh[None]

    h_ref[...] = h
    c_ref[...] = c
    hout_ref[...] = h
    cout_ref[...] = c


def _lstm2_body(acc_ref, cx_ref, wih_ref, whh_ref, b_ref, lens_ref,
                h0_ref, c0_ref, outc_ref, h_ref, c_ref):
    del acc_ref
    j = pl.program_id(0)

    @pl.when(j == 0)
    def _():
        h_ref[...] = h0_ref[...]
        c_ref[...] = c0_ref[...]

    h = h_ref[...]
    c = c_ref[...]
    whh = whh_ref[...]
    lens = lens_ref[...]

    gx = _dot16(cx_ref[...], wih_ref[...]) + b_ref[0]
    gx = gx.reshape(TK2, B, 4 * H)

    def sig(v):
        return 0.5 + 0.5 * jnp.tanh(0.5 * v)

    HB = B // 2
    ha, hb = h[:HB], h[HB:]
    ca, cb = c[:HB], c[HB:]

    def step(gx_s, h_half, c_half):
        gates = gx_s + _dot16(h_half, whh)
        i_g = sig(gates[:, 0 * H:1 * H])
        f_g = sig(gates[:, 1 * H:2 * H])
        g_g = jnp.tanh(gates[:, 2 * H:3 * H])
        o_g = sig(gates[:, 3 * H:4 * H])
        c_half = f_g * c_half + i_g * g_g
        h_half = o_g * jnp.tanh(c_half)
        return h_half, c_half

    for s in range(TK2):
        t = LQ + j * TK2 + s
        ha, ca = step(gx[s, :HB], ha, ca)
        hb, cb = step(gx[s, HB:], hb, cb)
        m = (lens > t).astype(jnp.float32)
        outc_ref[s] = jnp.concatenate([ha, hb], axis=0) * m

    h_ref[...] = jnp.concatenate([ha, hb], axis=0)
    c_ref[...] = jnp.concatenate([ca, cb], axis=0)


def _lstm(emb1, emb2, w_ih, w_hh, b, qproj_w, qproj_b, lens):
    RB = TK * B
    full = lambda shape: pl.BlockSpec(shape, lambda j: (0,) * len(shape))
    out_c1, out_q, h1, c1 = pl.pallas_call(
        _lstm1_body,
        grid=(LQ // TK,),
        in_specs=[
            pl.BlockSpec((RB, H), lambda j: (j, 0)),
            pl.BlockSpec((RB, H), lambda j: (LQ // TK + j, 0)),
            full((4 * H, H)),
            full((4 * H, H)),
            full((1, 4 * H)),
            full((H, H)),
            full((1, H)),
            full((2 * B, 1)),
        ],
        out_specs=[
            pl.BlockSpec((TK, B, H), lambda j: (j, 0, 0)),
            pl.BlockSpec((LQ, B, H), lambda j: (0, 0, 0)),
            full((2 * B, H)),
            full((2 * B, H)),
        ],
        out_shape=[
            jax.ShapeDtypeStruct((LC + 1, B, H), jnp.float32),
            jax.ShapeDtypeStruct((LQ, B, H), jnp.float32),
            jax.ShapeDtypeStruct((2 * B, H), jnp.float32),
            jax.ShapeDtypeStruct((2 * B, H), jnp.float32),
        ],
        scratch_shapes=[
            pltpu.VMEM((2 * B, H), jnp.float32),
            pltpu.VMEM((2 * B, H), jnp.float32),
        ],
        compiler_params=pltpu.CompilerParams(
            dimension_semantics=("arbitrary",),
        ),
    )(emb1, emb1, w_ih, w_hh, b, qproj_w, qproj_b, lens)

    RB2 = TK2 * B
    out_c = pl.pallas_call(
        _lstm2_body,
        grid=((LC - LQ) // TK2,),
        in_specs=[
            pl.BlockSpec(memory_space=pl.ANY),
            pl.BlockSpec((RB2, H), lambda j: (j, 0)),
            full((H, 4 * H)),
            full((H, 4 * H)),
            full((1, 4 * H)),
            full((B, 1)),
            pl.BlockSpec((B, H), lambda j: (0, 0)),
            pl.BlockSpec((B, H), lambda j: (0, 0)),
        ],
        out_specs=pl.BlockSpec((TK2, B, H), lambda j: (j + 1, 0, 0)),
        out_shape=jax.ShapeDtypeStruct((LC + 1, B, H), jnp.float32),
        scratch_shapes=[
            pltpu.VMEM((B, H), jnp.float32),
            pltpu.VMEM((B, H), jnp.float32),
        ],
        input_output_aliases={0: 0},
        compiler_params=pltpu.CompilerParams(
            dimension_semantics=("arbitrary",),
        ),
    )(out_c1, emb2, w_ih.T, w_hh.T, b, lens[:B], h1, c1)

    return out_c, out_q


def _sentinel_body(acc_ref, s_ref, out_ref):
    del acc_ref
    out_ref[...] = jnp.broadcast_to(s_ref[0], (1, B, H))


def _append_sentinel(out_c, sentinel):
    return pl.pallas_call(
        _sentinel_body,
        grid=(1,),
        in_specs=[
            pl.BlockSpec(memory_space=pl.ANY),
            pl.BlockSpec((1, H), lambda i: (0, 0)),
        ],
        out_specs=pl.BlockSpec((1, B, H), lambda i: (LC, 0, 0)),
        out_shape=jax.ShapeDtypeStruct((LC + 1, B, H), jnp.float32),
        input_output_aliases={0: 0},
    )(out_c, sentinel)


def kernel(cw_idxs, qw_idxs, word_vectors, proj_w, hwy_wg0, hwy_bg0,
           hwy_wt0, hwy_bt0, hwy_wg1, hwy_bg1, hwy_wt1, hwy_bt1,
           lstm_w_ih, lstm_w_hh, lstm_b, qproj_w, qproj_b,
           sentinel_c, sentinel_q):
    idx_c = cw_idxs.T.reshape(-1).astype(jnp.int32)
    idx_q = qw_idxs.T.reshape(-1).astype(jnp.int32)
    idx1 = jnp.concatenate([idx_c[:LQ * B], idx_q])
    pad = jnp.zeros((NW * 704 - (LC - LQ) * B,), jnp.int32)
    idx2 = jnp.concatenate([idx_c[LQ * B:], pad])

    table = _tablefwd(word_vectors.T, proj_w,
                      hwy_wg0, hwy_bg0.reshape(1, H), hwy_wt0, hwy_bt0.reshape(1, H),
                      hwy_wg1, hwy_bg1.reshape(1, H), hwy_wt1, hwy_bt1.reshape(1, H))

    emb1 = _sc_gather(table, idx1, 200, [128, 72])
    emb2 = _sc_gather(table, idx2, 704, [128] * 5 + [64])

    c_len = (cw_idxs != 0).sum(-1).astype(jnp.int32)
    q_len = (qw_idxs != 0).sum(-1).astype(jnp.int32)
    lens = jnp.concatenate([c_len, q_len]).reshape(2 * B, 1)

    b2 = lstm_b.reshape(1, 4 * H)
    qb2 = qproj_b.reshape(1, H)
    out_c, out_q = _lstm(emb1, emb2, lstm_w_ih, lstm_w_hh, b2,
                         qproj_w, qb2, lens)

    out_d = _append_sentinel(out_c, sentinel_c.reshape(1, H))
    D = jnp.transpose(out_d, (1, 0, 2))
    sq = jnp.broadcast_to(sentinel_q[None, None, :], (B, 1, H))
    Q = jnp.concatenate([jnp.transpose(out_q, (1, 0, 2)), sq], axis=1)
    return (D, Q)
